# baseline (speedup 1.0000x reference)
import functools

import jax
import jax.numpy as jnp
from jax import lax
from jax.experimental import pallas as pl
from jax.experimental.pallas import tpu as pltpu
from jax.experimental.pallas import tpu_sc as plsc

D = 64
G = 16384
NC = 2
NS = 16
NW = NC * NS
BPW = G // NW
L = 16
TR = 8
NGRP = BPW // L


def _build_sc_kernel():
    mesh = plsc.VectorSubcoreMesh(core_axis_name="c", subcore_axis_name="s")

    @functools.partial(
        pl.kernel,
        mesh=mesh,
        out_type=jax.ShapeDtypeStruct((NW, BPW, D), jnp.float32),
        scratch_types=[
            pltpu.VMEM((BPW,), jnp.int32),
            pltpu.VMEM((BPW,), jnp.int32),
            pltpu.VMEM((BPW,), jnp.int32),
            pltpu.VMEM((BPW,), jnp.float32),
            pltpu.VMEM((2, L, D), jnp.float32),
            pltpu.VMEM((512, D), jnp.float32),
            pltpu.VMEM((L, D), jnp.float32),
            pltpu.SemaphoreType.DMA,
            pltpu.SemaphoreType.DMA,
            pltpu.SemaphoreType.DMA,
        ],
    )
    def k(tidx_hbm, sub_hbm, midx_hbm, mask_hbm, pe_hbm, mt_hbm, out_hbm,
          tidx_v, sub_v, midx_v, mask_v, blk_v, mt_loc, out_v,
          semp0, semp1, semm):
        wid = lax.axis_index("s") * NC + lax.axis_index("c")
        semp = (semp0, semp1)

        mt_cp = pltpu.async_copy(mt_hbm, mt_loc, semm)
        pltpu.sync_copy(tidx_hbm.at[wid], tidx_v)
        pltpu.sync_copy(sub_hbm.at[wid], sub_v)
        pltpu.sync_copy(midx_hbm.at[wid], midx_v)
        pltpu.sync_copy(mask_hbm.at[wid], mask_v)

        def fire(g, slot):
            tvec = tidx_v[pl.ds(g * L, L)]
            rvec = sub_v[pl.ds(g * L, L)]
            for k in range(L):
                pltpu.async_copy(pe_hbm.at[tvec[k], rvec[k]],
                                 blk_v.at[slot, k], semp[slot])

        def wait(slot):
            for k in range(L):
                pltpu.make_async_copy(pe_hbm.at[0, 0], blk_v.at[slot, k],
                                      semp[slot]).wait()

        def extract(g, slot):
            mvec = mask_v[pl.ds(g * L, L)]
            mivec = midx_v[pl.ds(g * L, L)]
            for k in range(L):
                m = mvec[k]
                mi = mivec[k]
                for j in range(D // L):
                    sl = pl.ds(L * j, L)
                    pe = blk_v[slot, k, sl]
                    ms = mt_loc[mi, sl]
                    out_v[k, sl] = m * (pe - ms) + ms
            pltpu.sync_copy(out_v, out_hbm.at[wid, pl.ds(g * L, L)])

        fire(0, 0)
        mt_cp.wait()

        def pair(p, carry):
            ga = 2 * p
            fire(ga + 1, 1)
            wait(0)
            extract(ga, 0)

            @pl.when(p + 1 < NGRP // 2)
            def _():
                fire(ga + 2, 0)

            wait(1)
            extract(ga + 1, 1)
            return carry

        lax.fori_loop(0, NGRP // 2, pair, 0)

    return k


@jax.jit
def kernel(indices, present_mask, missing_idx_map, pe_table, missing_table):
    idx = indices.astype(jnp.int32)
    tidx = (idx // TR).reshape(NW, BPW)
    sub = (idx % TR).reshape(NW, BPW)
    midx = missing_idx_map.astype(jnp.int32).reshape(NW, BPW)
    mask = present_mask.astype(jnp.float32).reshape(NW, BPW)
    pe3 = pe_table.reshape(pe_table.shape[0] // TR, TR, D)
    out = _build_sc_kernel()(tidx, sub, midx, mask, pe3, missing_table)
    return out.reshape(G, D)


# speedup vs baseline: 1.0298x; 1.0298x over previous
import functools

import jax
import jax.numpy as jnp
from jax import lax
from jax.experimental import pallas as pl
from jax.experimental.pallas import tpu as pltpu
from jax.experimental.pallas import tpu_sc as plsc

D = 64
G = 16384
NC = 2
NS = 16
NW = NC * NS
BPW = G // NW
L = 16
TR = 8
NGRP = BPW // L


def _build_sc_kernel():
    mesh = plsc.VectorSubcoreMesh(core_axis_name="c", subcore_axis_name="s")

    @functools.partial(
        pl.kernel,
        mesh=mesh,
        out_type=jax.ShapeDtypeStruct((NW, BPW, D), jnp.float32),
        scratch_types=[
            pltpu.VMEM((BPW,), jnp.int32),
            pltpu.VMEM((BPW,), jnp.int32),
            pltpu.VMEM((BPW,), jnp.int32),
            pltpu.VMEM((BPW,), jnp.float32),
            pltpu.VMEM((2, L, D), jnp.float32),
            pltpu.VMEM((2, L, 128), jnp.float32),
            pltpu.VMEM((2, L, D), jnp.float32),
            pltpu.SemaphoreType.DMA,
            pltpu.SemaphoreType.DMA,
            pltpu.SemaphoreType.DMA,
            pltpu.SemaphoreType.DMA,
            pltpu.SemaphoreType.DMA,
            pltpu.SemaphoreType.DMA,
        ],
    )
    def k(tidx_hbm, sub_hbm, midx_hbm, mask_hbm, pe_hbm, mt_hbm, out_hbm,
          tidx_v, sub_v, midx_v, mask_v, blk_v, ms_v, out_v,
          semp0, semp1, semm0, semm1, semo0, semo1):
        wid = lax.axis_index("s") * NC + lax.axis_index("c")
        semp = (semp0, semp1)
        semm = (semm0, semm1)
        semo = (semo0, semo1)

        pltpu.sync_copy(tidx_hbm.at[wid], tidx_v)
        pltpu.sync_copy(sub_hbm.at[wid], sub_v)
        pltpu.sync_copy(midx_hbm.at[wid], midx_v)
        pltpu.sync_copy(mask_hbm.at[wid], mask_v)

        def fire(g, slot):
            tvec = tidx_v[pl.ds(g * L, L)]
            rvec = sub_v[pl.ds(g * L, L)]
            for k in range(L):
                pltpu.async_copy(pe_hbm.at[tvec[k], rvec[k]],
                                 blk_v.at[slot, k], semp[slot])
            pltpu.async_copy(mt_hbm.at[midx_v.at[pl.ds(g * L, L)]],
                             ms_v.at[slot], semm[slot])

        def wait(slot):
            for k in range(L):
                pltpu.make_async_copy(pe_hbm.at[0, 0], blk_v.at[slot, k],
                                      semp[slot]).wait()
            pltpu.make_async_copy(mt_hbm.at[midx_v.at[pl.ds(0, L)]],
                                  ms_v.at[slot], semm[slot]).wait()

        def extract(g, slot, first):
            mvec = mask_v[pl.ds(g * L, L)]
            if not first:
                pltpu.make_async_copy(
                    out_v.at[slot], out_hbm.at[wid, pl.ds(0, L)],
                    semo[slot]).wait()
            for k in range(L):
                m = mvec[k]
                for j in range(D // L):
                    sl = pl.ds(L * j, L)
                    pe = blk_v[slot, k, sl]
                    ms = ms_v[slot, k, sl]
                    out_v[slot, k, sl] = m * (pe - ms) + ms
            pltpu.async_copy(out_v.at[slot], out_hbm.at[wid, pl.ds(g * L, L)],
                             semo[slot])

        fire(0, 0)

        fire(1, 1)
        wait(0)
        extract(0, 0, True)
        fire(2, 0)
        wait(1)
        extract(1, 1, True)

        def pair2(p, carry):
            ga = 2 * p
            fire(ga + 1, 1)
            wait(0)
            extract(ga, 0, False)

            @pl.when(p + 1 < NGRP // 2)
            def _():
                fire(ga + 2, 0)

            wait(1)
            extract(ga + 1, 1, False)
            return carry

        lax.fori_loop(1, NGRP // 2, pair2, 0)
        pltpu.make_async_copy(out_v.at[0], out_hbm.at[wid, pl.ds(0, L)],
                              semo[0]).wait()
        pltpu.make_async_copy(out_v.at[1], out_hbm.at[wid, pl.ds(0, L)],
                              semo[1]).wait()

    return k


@jax.jit
def kernel(indices, present_mask, missing_idx_map, pe_table, missing_table):
    idx = indices.astype(jnp.int32)
    tidx = (idx // TR).reshape(NW, BPW)
    sub = (idx % TR).reshape(NW, BPW)
    midx = missing_idx_map.astype(jnp.int32).reshape(NW, BPW)
    mask = present_mask.astype(jnp.float32).reshape(NW, BPW)
    n_missing = missing_table.shape[0]
    mt_ext = jnp.zeros((n_missing, 128), jnp.float32)
    mt_ext = lax.dynamic_update_slice(
        mt_ext, missing_table.astype(jnp.float32), (0, 0))
    pe3 = pe_table.reshape(pe_table.shape[0] // TR, TR, D)
    out = _build_sc_kernel()(tidx, sub, midx, mask, pe3, mt_ext)
    return out.reshape(G, D)


# speedup vs baseline: 1.0464x; 1.0161x over previous
import functools

import jax
import jax.numpy as jnp
from jax import lax
from jax.experimental import pallas as pl
from jax.experimental.pallas import tpu as pltpu
from jax.experimental.pallas import tpu_sc as plsc

D = 64
G = 16384
NC = 2
NS = 16
NW = NC * NS
BPW = G // NW
L = 16
TR = 8
NGRP = BPW // L


def _build_sc_kernel():
    mesh = plsc.VectorSubcoreMesh(core_axis_name="c", subcore_axis_name="s")

    @functools.partial(
        pl.kernel,
        mesh=mesh,
        out_type=jax.ShapeDtypeStruct((NW, BPW, D), jnp.float32),
        scratch_types=[
            pltpu.VMEM((BPW,), jnp.int32),
            pltpu.VMEM((BPW,), jnp.int32),
            pltpu.VMEM((BPW,), jnp.int32),
            pltpu.VMEM((BPW,), jnp.float32),
            pltpu.VMEM((4, L, D), jnp.float32),
            pltpu.VMEM((4, L, 128), jnp.float32),
            pltpu.VMEM((L, D), jnp.float32),
            pltpu.SemaphoreType.DMA,
            pltpu.SemaphoreType.DMA,
            pltpu.SemaphoreType.DMA,
            pltpu.SemaphoreType.DMA,
            pltpu.SemaphoreType.DMA,
            pltpu.SemaphoreType.DMA,
            pltpu.SemaphoreType.DMA,
            pltpu.SemaphoreType.DMA,
        ],
    )
    def k(tidx_hbm, sub_hbm, midx_hbm, mask_hbm, pe_hbm, mt_hbm, out_hbm,
          tidx_v, sub_v, midx_v, mask_v, blk_v, ms_v, out_v,
          semp0, semp1, semp2, semp3, semm0, semm1, semm2, semm3):
        wid = lax.axis_index("s") * NC + lax.axis_index("c")
        semp = (semp0, semp1, semp2, semp3)
        semm = (semm0, semm1, semm2, semm3)

        pltpu.sync_copy(tidx_hbm.at[wid], tidx_v)
        pltpu.sync_copy(sub_hbm.at[wid], sub_v)
        pltpu.sync_copy(midx_hbm.at[wid], midx_v)
        pltpu.sync_copy(mask_hbm.at[wid], mask_v)

        def fire(g, slot):
            tvec = tidx_v[pl.ds(g * L, L)]
            rvec = sub_v[pl.ds(g * L, L)]
            for k in range(L):
                pltpu.async_copy(pe_hbm.at[tvec[k], rvec[k]],
                                 blk_v.at[slot, k], semp[slot])
            pltpu.async_copy(mt_hbm.at[midx_v.at[pl.ds(g * L, L)]],
                             ms_v.at[slot], semm[slot])

        def wait(slot):
            for k in range(L):
                pltpu.make_async_copy(pe_hbm.at[0, 0], blk_v.at[slot, k],
                                      semp[slot]).wait()
            pltpu.make_async_copy(mt_hbm.at[midx_v.at[pl.ds(0, L)]],
                                  ms_v.at[slot], semm[slot]).wait()

        def extract(g, slot):
            mvec = mask_v[pl.ds(g * L, L)]
            for k in range(L):
                m = mvec[k]
                for j in range(D // L):
                    sl = pl.ds(L * j, L)
                    pe = blk_v[slot, k, sl]
                    ms = ms_v[slot, k, sl]
                    out_v[k, sl] = m * (pe - ms) + ms
            pltpu.sync_copy(out_v, out_hbm.at[wid, pl.ds(g * L, L)])

        fire(0, 0)
        fire(1, 1)
        fire(2, 2)

        def quad(p, carry):
            gb = 4 * p
            for q in range(4):
                g = gb + q

                @pl.when(g + 3 < NGRP)
                def _():
                    fire(g + 3, (q + 3) % 4)

                wait(q)
                extract(g, q)
            return carry

        lax.fori_loop(0, NGRP // 4, quad, 0)

    return k


@jax.jit
def kernel(indices, present_mask, missing_idx_map, pe_table, missing_table):
    idx = indices.astype(jnp.int32)
    tidx = (idx // TR).reshape(NW, BPW)
    sub = (idx % TR).reshape(NW, BPW)
    midx = missing_idx_map.astype(jnp.int32).reshape(NW, BPW)
    mask = present_mask.astype(jnp.float32).reshape(NW, BPW)
    n_missing = missing_table.shape[0]
    mt_ext = jnp.zeros((n_missing, 128), jnp.float32)
    mt_ext = lax.dynamic_update_slice(
        mt_ext, missing_table.astype(jnp.float32), (0, 0))
    pe3 = pe_table.reshape(pe_table.shape[0] // TR, TR, D)
    out = _build_sc_kernel()(tidx, sub, midx, mask, pe3, mt_ext)
    return out.reshape(G, D)
